# Initial kernel scaffold; baseline (speedup 1.0000x reference)
import jax
import jax.numpy as jnp
from jax.experimental import pallas as pl


def kernel(x):
    raise NotImplementedError("write your pallas kernel here")


# speedup vs baseline: 1.0587x; 1.0587x over previous
import math

import jax
import jax.numpy as jnp
from jax.experimental import pallas as pl
from jax.experimental.pallas import tpu as pltpu

_R_RATIO = 0.95


def _scores_maxarg(a_h, b_h, bm, bn):
    B, T2, C = a_h.shape

    def body(a_ref, b_ref, nm_ref, ni_ref):
        j = pl.program_id(2)
        an = a_ref[0].astype(jnp.bfloat16)
        bn_ = b_ref[0].astype(jnp.bfloat16)
        s = jax.lax.dot_general(an, bn_, (((1,), (1,)), ((), ())),
                                preferred_element_type=jnp.float32)
        blk_max = jnp.max(s, axis=1)
        m = blk_max[:, None]
        iota = jax.lax.broadcasted_iota(jnp.int32, s.shape, 1)
        blk_arg = jnp.min(jnp.where(s == m, iota, T2), axis=1) + j * bn

        @pl.when(j == 0)
        def _():
            nm_ref[0, 0, :] = blk_max
            ni_ref[0, 0, :] = blk_arg

        @pl.when(j != 0)
        def _():
            cur = nm_ref[0, 0, :]
            take = blk_max > cur
            nm_ref[0, 0, :] = jnp.where(take, blk_max, cur)
            ni_ref[0, 0, :] = jnp.where(take, blk_arg, ni_ref[0, 0, :])

    grid = (B, T2 // bm, T2 // bn)
    nm, ni = pl.pallas_call(
        body,
        grid=grid,
        in_specs=[
            pl.BlockSpec((1, bm, C), lambda b, i, j: (b, i, 0)),
            pl.BlockSpec((1, bn, C), lambda b, i, j: (b, j, 0)),
        ],
        out_specs=[
            pl.BlockSpec((1, 1, bm), lambda b, i, j: (b, 0, i)),
            pl.BlockSpec((1, 1, bm), lambda b, i, j: (b, 0, i)),
        ],
        out_shape=[
            jax.ShapeDtypeStruct((B, 1, T2), jnp.float32),
            jax.ShapeDtypeStruct((B, 1, T2), jnp.int32),
        ],
        compiler_params=pltpu.CompilerParams(
            dimension_semantics=("parallel", "parallel", "arbitrary"),
        ),
    )(a_h, b_h)
    return nm, ni


def _ranks(nm, bm):
    B, _, T2 = nm.shape

    def body(vi_ref, vj_ref, out_ref):
        i = pl.program_id(1)
        vi = vi_ref[0, 0, :][:, None]
        vj = vj_ref[0, 0, :][None, :]
        jg = jax.lax.broadcasted_iota(jnp.int32, (bm, T2), 1)
        ig = jax.lax.broadcasted_iota(jnp.int32, (bm, T2), 0) + i * bm
        gt = vj > vi
        eq_before = (vj == vi) & (jg < ig)
        out_ref[0, 0, :] = jnp.sum((gt | eq_before).astype(jnp.int32), axis=1)

    rank = pl.pallas_call(
        body,
        grid=(B, T2 // bm),
        in_specs=[
            pl.BlockSpec((1, 1, bm), lambda b, i: (b, 0, i)),
            pl.BlockSpec((1, 1, T2), lambda b, i: (b, 0, 0)),
        ],
        out_specs=pl.BlockSpec((1, 1, bm), lambda b, i: (b, 0, i)),
        out_shape=jax.ShapeDtypeStruct((B, 1, T2), jnp.int32),
        compiler_params=pltpu.CompilerParams(
            dimension_semantics=("parallel", "arbitrary"),
        ),
    )(nm, nm)
    return rank


def _merge(a_h, b_h, rank, node_idx, r, bq):
    B, T2, C = a_h.shape
    nq = 2 * T2 // bq

    def body(rank_ref, nidx_ref, src_ref, dst_ref, out_ref):
        q = pl.program_id(1)
        rk = rank_ref[0, 0, :][None, :]
        ni = nidx_ref[0, 0, :][None, :]
        qg = jax.lax.broadcasted_iota(jnp.int32, (bq, T2), 0) + q * bq
        is_dst = qg >= T2
        p_unm = rk == (r + qg)
        p_dst = (rk < r) & (ni == (qg - T2))
        pmat = ((is_dst & p_dst) | (~is_dst & p_unm)).astype(jnp.float32)
        upd = jnp.dot(pmat, src_ref[0], preferred_element_type=jnp.float32,
                      precision=jax.lax.Precision.HIGHEST)
        cnt = jnp.sum(pmat, axis=1)
        row_is_dst = is_dst[:, 0]
        base = jnp.where(row_is_dst[:, None], dst_ref[0], 0.0)
        denom = jnp.where(row_is_dst, 1.0 + cnt, jnp.maximum(cnt, 1.0))
        out_ref[0] = (base + upd) / denom[:, None]

    def dst_map(b, q):
        nq_half = T2 // bq
        return (b, jnp.maximum(q - nq_half, 0), 0)

    merged = pl.pallas_call(
        body,
        grid=(B, nq),
        in_specs=[
            pl.BlockSpec((1, 1, T2), lambda b, q: (b, 0, 0)),
            pl.BlockSpec((1, 1, T2), lambda b, q: (b, 0, 0)),
            pl.BlockSpec((1, T2, C), lambda b, q: (b, 0, 0)),
            pl.BlockSpec((1, bq, C), dst_map),
        ],
        out_specs=pl.BlockSpec((1, bq, C), lambda b, q: (b, q, 0)),
        out_shape=jax.ShapeDtypeStruct((B, 2 * T2, C), jnp.float32),
        compiler_params=pltpu.CompilerParams(
            dimension_semantics=("parallel", "parallel"),
        ),
    )(rank, node_idx, a_h, b_h)
    return merged


def kernel(x):
    B, T, C = x.shape
    T2 = T // 2
    r = math.floor(T - T * _R_RATIO)
    n = jnp.linalg.norm(x, axis=-1, keepdims=True)
    xn = x / jnp.clip(n, 1e-12)
    a_h = xn[:, ::2, :]
    b_h = xn[:, 1::2, :]
    a_raw = x[:, ::2, :]
    b_raw = x[:, 1::2, :]

    bm1 = min(512, T2)
    bn1 = min(1024, T2)
    nm, ni = _scores_maxarg(a_h, b_h, bm1, bn1)
    rank = _ranks(nm, min(512, T2))
    merged = _merge(a_raw, b_raw, rank, ni, r, min(256, T2))
    return jnp.concatenate([merged[:, : T2 - r], merged[:, T2:]], axis=1)


# speedup vs baseline: 2.0442x; 1.9308x over previous
import functools
import math

import jax
import jax.numpy as jnp
from jax import lax
from jax.experimental import pallas as pl
from jax.experimental.pallas import tpu as pltpu
from jax.experimental.pallas import tpu_sc as plsc

_R_RATIO = 0.95
_NW = 32
_SPW = 13


def _scores_maxarg(a_h, b_h, bm, bn):
    B, T2, C = a_h.shape

    def body(a_ref, b_ref, nm_ref, ni_ref):
        j = pl.program_id(2)
        an = a_ref[0].astype(jnp.bfloat16)
        bn_ = b_ref[0].astype(jnp.bfloat16)
        s = lax.dot_general(an, bn_, (((1,), (1,)), ((), ())),
                            preferred_element_type=jnp.float32)
        blk_max = jnp.max(s, axis=1)
        m = blk_max[:, None]
        iota = lax.broadcasted_iota(jnp.int32, s.shape, 1)
        blk_arg = jnp.min(jnp.where(s == m, iota, T2), axis=1) + j * bn

        @pl.when(j == 0)
        def _():
            nm_ref[0, 0, :] = blk_max
            ni_ref[0, 0, :] = blk_arg

        @pl.when(j != 0)
        def _():
            cur = nm_ref[0, 0, :]
            take = blk_max > cur
            nm_ref[0, 0, :] = jnp.where(take, blk_max, cur)
            ni_ref[0, 0, :] = jnp.where(take, blk_arg, ni_ref[0, 0, :])

    grid = (B, T2 // bm, T2 // bn)
    nm, ni = pl.pallas_call(
        body,
        grid=grid,
        in_specs=[
            pl.BlockSpec((1, bm, C), lambda b, i, j: (b, i, 0)),
            pl.BlockSpec((1, bn, C), lambda b, i, j: (b, j, 0)),
        ],
        out_specs=[
            pl.BlockSpec((1, 1, bm), lambda b, i, j: (b, 0, i)),
            pl.BlockSpec((1, 1, bm), lambda b, i, j: (b, 0, i)),
        ],
        out_shape=[
            jax.ShapeDtypeStruct((B, 1, T2), jnp.float32),
            jax.ShapeDtypeStruct((B, 1, T2), jnp.int32),
        ],
        compiler_params=pltpu.CompilerParams(
            dimension_semantics=("parallel", "parallel", "arbitrary"),
        ),
    )(a_h, b_h)
    return nm, ni


def _ranks(nm, bm):
    B, _, T2 = nm.shape

    def body(vi_ref, vj_ref, out_ref):
        i = pl.program_id(1)
        vi = vi_ref[0, 0, :][:, None]
        vj = vj_ref[0, 0, :][None, :]
        jg = lax.broadcasted_iota(jnp.int32, (bm, T2), 1)
        ig = lax.broadcasted_iota(jnp.int32, (bm, T2), 0) + i * bm
        gt = vj > vi
        eq_before = (vj == vi) & (jg < ig)
        out_ref[0, 0, :] = jnp.sum((gt | eq_before).astype(jnp.int32), axis=1)

    rank = pl.pallas_call(
        body,
        grid=(B, T2 // bm),
        in_specs=[
            pl.BlockSpec((1, 1, bm), lambda b, i: (b, 0, i)),
            pl.BlockSpec((1, 1, T2), lambda b, i: (b, 0, 0)),
        ],
        out_specs=pl.BlockSpec((1, 1, bm), lambda b, i: (b, 0, i)),
        out_shape=jax.ShapeDtypeStruct((B, 1, T2), jnp.int32),
        compiler_params=pltpu.CompilerParams(
            dimension_semantics=("parallel", "arbitrary"),
        ),
    )(nm, nm)
    return rank


def _perm_scatter(rank_ref, rank_v, perm_v, b, T2, i16):
    pltpu.sync_copy(rank_ref.at[pl.ds(b * T2, T2)], rank_v)

    def pbody(i, carry):
        kv = plsc.load_gather(rank_v, [i16 + i * 16])
        plsc.store_scatter(perm_v, [kv], i16 + i * 16)
        return carry
    lax.fori_loop(0, T2 // 16, pbody, 0, unroll=8)


def _sc_slots(x_flat, rank_flat, nidx_flat, B, T2, C, r):
    mesh = plsc.VectorSubcoreMesh(core_axis_name="c", subcore_axis_name="s")

    @functools.partial(
        pl.kernel,
        out_type=[
            jax.ShapeDtypeStruct((B, _NW, 16, C), jnp.float32),
            jax.ShapeDtypeStruct((B, _NW, 16, C), jnp.float32),
            jax.ShapeDtypeStruct((B, _NW, 16), jnp.int32),
        ],
        mesh=mesh,
        compiler_params=pltpu.CompilerParams(needs_layout_passes=False),
        scratch_types=[
            pltpu.VMEM((T2,), jnp.int32),
            pltpu.VMEM((T2,), jnp.int32),
            pltpu.VMEM((T2,), jnp.int32),
            pltpu.VMEM((16,), jnp.int32),
            pltpu.VMEM((16,), jnp.int32),
            pltpu.VMEM((16, C), jnp.float32),
            pltpu.SemaphoreType.DMA,
        ],
    )
    def sc_kernel(x_ref, rank_ref, nidx_ref, srcg_ref, dstg_ref, dselp_ref,
                  rank_v, nidx_v, perm_v, idx16_v, dsel_v, rows16_v, sem):
        wid = lax.axis_index("s") * 2 + lax.axis_index("c")
        i16 = lax.broadcasted_iota(jnp.int32, (16,), 0)
        for b in range(B):
            base_x = b * 2 * T2
            pltpu.sync_copy(nidx_ref.at[pl.ds(b * T2, T2)], nidx_v)
            _perm_scatter(rank_ref, rank_v, perm_v, b, T2, i16)

            sbase = wid * _SPW
            permk = plsc.load_gather(perm_v, [i16 + sbase])
            dselv = plsc.load_gather(nidx_v, [permk])
            p408 = plsc.load_gather(perm_v, [jnp.full((16,), r - 1, jnp.int32)])
            d408 = plsc.load_gather(nidx_v, [p408])
            valid = (i16 < _SPW) & ((sbase + i16) < r)
            dselv = jnp.where(valid, dselv, d408)
            dsel_v[...] = dselv
            pltpu.sync_copy(dsel_v, dselp_ref.at[b, wid])
            idx16_v[...] = permk * 2 + base_x
            pltpu.async_copy(x_ref.at[idx16_v], rows16_v, sem).wait()
            pltpu.sync_copy(rows16_v, srcg_ref.at[b, wid])
            idx16_v[...] = dselv * 2 + (base_x + 1)
            pltpu.async_copy(x_ref.at[idx16_v], rows16_v, sem).wait()
            pltpu.sync_copy(rows16_v, dstg_ref.at[b, wid])

    return sc_kernel(x_flat, rank_flat, nidx_flat)


def _compact_merge(srcg, dstg, dselp, r):
    B = srcg.shape[0]
    S = _NW * 16
    C = srcg.shape[-1]
    src2 = srcg.reshape(B, S, C)
    dst2 = dstg.reshape(B, S, C)
    dsel2 = dselp.reshape(B, 1, S)

    def body(dsel_ref, src_ref, dst_ref, out_ref):
        dse = dsel_ref[0, 0, :]
        kp = lax.broadcasted_iota(jnp.int32, (S, S), 1)
        lane = kp & 15
        gslot = (kp >> 4) * _SPW + lane
        kvalid = (lane < _SPW) & (gslot < r)
        member = (dse[:, None] == dse[None, :]) & kvalid
        mf = member.astype(jnp.float32)
        upd = jnp.dot(mf, src_ref[0], preferred_element_type=jnp.float32,
                      precision=lax.Precision.HIGHEST)
        cnt = jnp.sum(mf, axis=1)
        out_ref[0] = (dst_ref[0] + upd) / (1.0 + cnt)[:, None]

    return pl.pallas_call(
        body,
        grid=(B,),
        in_specs=[
            pl.BlockSpec((1, 1, S), lambda b: (b, 0, 0)),
            pl.BlockSpec((1, S, C), lambda b: (b, 0, 0)),
            pl.BlockSpec((1, S, C), lambda b: (b, 0, 0)),
        ],
        out_specs=pl.BlockSpec((1, S, C), lambda b: (b, 0, 0)),
        out_shape=jax.ShapeDtypeStruct((B, S, C), jnp.float32),
    )(dsel2, src2, dst2)


def _sc_finalize(out_ref_arg, x_flat, rank_flat, dsel512, dselp3, merged4,
                 B, T2, C, r, tout):
    nunm = T2 - r
    UPW = -(-nunm // _NW)
    DPW = T2 // _NW
    mesh = plsc.VectorSubcoreMesh(core_axis_name="c", subcore_axis_name="s")

    @functools.partial(
        pl.kernel,
        out_type=[],
        mesh=mesh,
        compiler_params=pltpu.CompilerParams(needs_layout_passes=False),
        scratch_types=[
            pltpu.VMEM((T2,), jnp.int32),
            pltpu.VMEM((T2,), jnp.int32),
            pltpu.VMEM((T2,), jnp.int32),
            pltpu.VMEM((512,), jnp.int32),
            pltpu.VMEM((128,), jnp.int32),
            pltpu.VMEM((128,), jnp.int32),
            pltpu.VMEM((128, C), jnp.float32),
            pltpu.SemaphoreType.DMA,
        ],
    )
    def sc_kernel(x_ref, rank_ref, dsel_ref, out_ref,
                  rank_v, perm_v, touched_v, dsel_v, idx_v, widx_v,
                  rows_v, sem):
        wid = lax.axis_index("s") * 2 + lax.axis_index("c")
        i16 = lax.broadcasted_iota(jnp.int32, (16,), 0)
        zero16 = jnp.zeros((16,), jnp.int32)
        one16 = jnp.ones((16,), jnp.int32)
        for b in range(B):
            base_x = b * 2 * T2
            base_out = b * tout
            _perm_scatter(rank_ref, rank_v, perm_v, b, T2, i16)
            pltpu.sync_copy(dsel_ref.at[pl.ds(b * 512, 512)], dsel_v)

            def zbody(i, carry):
                plsc.store_scatter(touched_v, [i16 + i * 16], zero16)
                return carry
            lax.fori_loop(0, T2 // 16, zbody, 0, unroll=8)

            def tbody(i, carry):
                dchunk = plsc.load_gather(dsel_v, [i16 + i * 16])
                kvalid = (i16 < _SPW) & ((i * _SPW + i16) < r)
                plsc.store_scatter(touched_v, [dchunk], one16, mask=kvalid)
                return carry
            lax.fori_loop(0, 32, tbody, 0, unroll=4)

            s408 = (r - 1) // _SPW * 16 + (r - 1) % _SPW
            d408 = plsc.load_gather(dsel_v, [jnp.full((16,), s408, jnp.int32)])
            midx = base_out + nunm + d408

            def ubody(i, carry):
                j = i * 16 + i16
                uw = wid * UPW + j
                q = jnp.minimum(r + uw, T2 - 1)
                pv = plsc.load_gather(perm_v, [q])
                plsc.store_scatter(idx_v, [j], pv * 2 + base_x)
                ok = (j < UPW) & (uw < nunm)
                plsc.store_scatter(widx_v, [j],
                                   jnp.where(ok, base_out + uw, midx))
                return carry
            lax.fori_loop(0, 8, ubody, 0, unroll=8)
            pltpu.async_copy(x_ref.at[idx_v], rows_v, sem).wait()
            pltpu.async_copy(rows_v, out_ref.at[widx_v], sem).wait()

            def dbody(i, carry):
                j = i * 16 + i16
                d = wid * DPW + j
                plsc.store_scatter(idx_v, [j], d * 2 + (base_x + 1))
                tfl = plsc.load_gather(touched_v, [d])
                plsc.store_scatter(widx_v, [j],
                                   jnp.where(tfl == 0,
                                             base_out + nunm + d, midx))
                return carry
            lax.fori_loop(0, DPW // 16, dbody, 0, unroll=8)
            pltpu.async_copy(x_ref.at[idx_v], rows_v, sem).wait()
            pltpu.async_copy(rows_v, out_ref.at[widx_v], sem).wait()


    sc_kernel(x_flat, rank_flat, dsel512, out_ref_arg)

    @functools.partial(
        pl.kernel,
        out_type=[],
        mesh=mesh,
        compiler_params=pltpu.CompilerParams(needs_layout_passes=False),
        scratch_types=[
            pltpu.VMEM((16,), jnp.int32),
            pltpu.VMEM((16,), jnp.int32),
            pltpu.VMEM((16, C), jnp.float32),
            pltpu.SemaphoreType.DMA,
        ],
    )
    def sc_scatter(dselp_ref, merged_ref, out_ref, dsel16_v, midx_v,
                   rows16_v, sem):
        wid = lax.axis_index("s") * 2 + lax.axis_index("c")
        for b in range(B):
            base_out = b * tout
            pltpu.sync_copy(dselp_ref.at[b, wid], dsel16_v)
            midx_v[...] = dsel16_v[...] + (base_out + nunm)
            pltpu.sync_copy(merged_ref.at[b, wid], rows16_v)
            pltpu.async_copy(rows16_v, out_ref.at[midx_v], sem).wait()

    sc_scatter(dselp3, merged4, out_ref_arg)


def kernel(x):
    B, T, C = x.shape
    T2 = T // 2
    r = math.floor(T - T * _R_RATIO)
    nunm = T2 - r
    tout = nunm + T2
    n = jnp.linalg.norm(x, axis=-1, keepdims=True)
    xn = x / jnp.clip(n, 1e-12)
    a_h = xn[:, ::2, :]
    b_h = xn[:, 1::2, :]

    nm, ni = _scores_maxarg(a_h, b_h, min(512, T2), min(1024, T2))
    rank = _ranks(nm, min(512, T2))

    x_flat = x.reshape(B * T, C)
    rank_flat = rank.reshape(B * T2)
    srcg, dstg, dselp = _sc_slots(x_flat, rank_flat, ni.reshape(B * T2),
                                  B, T2, C, r)
    merged = _compact_merge(srcg, dstg, dselp, r)

    out_ref = jax.new_ref(jnp.zeros((B * tout, C), jnp.float32))
    _sc_finalize(out_ref, x_flat, rank_flat, dselp.reshape(B * 512), dselp,
                 merged.reshape(B, _NW, 16, C), B, T2, C, r, tout)
    return out_ref[...].reshape(B, tout, C)


# speedup vs baseline: 2.7727x; 1.3564x over previous
import functools
import math

import jax
import jax.numpy as jnp
from jax import lax
from jax.experimental import pallas as pl
from jax.experimental.pallas import tpu as pltpu
from jax.experimental.pallas import tpu_sc as plsc

_R_RATIO = 0.95
_NW = 32
_SPW = 13


def _deinterleave(xn, bm):
    B, T, C = xn.shape
    T2 = T // 2

    def body(x_ref, a_ref, b_ref):
        blk = x_ref[0].reshape(-1, 2, x_ref.shape[-1])
        a_ref[0] = blk[:, 0, :]
        b_ref[0] = blk[:, 1, :]

    return pl.pallas_call(
        body,
        grid=(B, T2 // bm),
        in_specs=[pl.BlockSpec((1, 2 * bm, C), lambda b, i: (b, i, 0))],
        out_specs=[
            pl.BlockSpec((1, bm, C), lambda b, i: (b, i, 0)),
            pl.BlockSpec((1, bm, C), lambda b, i: (b, i, 0)),
        ],
        out_shape=[
            jax.ShapeDtypeStruct((B, T2, C), jnp.float32),
            jax.ShapeDtypeStruct((B, T2, C), jnp.float32),
        ],
    )(xn)


def _scores_maxarg(a_h, b_h, bm, bn):
    B, T2, C = a_h.shape

    def body(a_ref, b_ref, nm_ref, ni_ref):
        j = pl.program_id(2)
        an = a_ref[0].astype(jnp.bfloat16)
        bn_ = b_ref[0].astype(jnp.bfloat16)
        s = lax.dot_general(an, bn_, (((1,), (1,)), ((), ())),
                            preferred_element_type=jnp.float32)
        blk_max = jnp.max(s, axis=1)
        m = blk_max[:, None]
        iota = lax.broadcasted_iota(jnp.int32, s.shape, 1)
        blk_arg = jnp.min(jnp.where(s == m, iota, T2), axis=1) + j * bn

        @pl.when(j == 0)
        def _():
            nm_ref[0, 0, :] = blk_max
            ni_ref[0, 0, :] = blk_arg

        @pl.when(j != 0)
        def _():
            cur = nm_ref[0, 0, :]
            take = blk_max > cur
            nm_ref[0, 0, :] = jnp.where(take, blk_max, cur)
            ni_ref[0, 0, :] = jnp.where(take, blk_arg, ni_ref[0, 0, :])

    grid = (B, T2 // bm, T2 // bn)
    nm, ni = pl.pallas_call(
        body,
        grid=grid,
        in_specs=[
            pl.BlockSpec((1, bm, C), lambda b, i, j: (b, i, 0)),
            pl.BlockSpec((1, bn, C), lambda b, i, j: (b, j, 0)),
        ],
        out_specs=[
            pl.BlockSpec((1, 1, bm), lambda b, i, j: (b, 0, i)),
            pl.BlockSpec((1, 1, bm), lambda b, i, j: (b, 0, i)),
        ],
        out_shape=[
            jax.ShapeDtypeStruct((B, 1, T2), jnp.float32),
            jax.ShapeDtypeStruct((B, 1, T2), jnp.int32),
        ],
        compiler_params=pltpu.CompilerParams(
            dimension_semantics=("parallel", "parallel", "arbitrary"),
        ),
    )(a_h, b_h)
    return nm, ni


def _ranks(nm, bm):
    B, _, T2 = nm.shape

    def body(vi_ref, vj_ref, out_ref):
        i = pl.program_id(1)
        vi = vi_ref[0, 0, :][:, None]
        vj = vj_ref[0, 0, :][None, :]
        jg = lax.broadcasted_iota(jnp.int32, (bm, T2), 1)
        ig = lax.broadcasted_iota(jnp.int32, (bm, T2), 0) + i * bm
        gt = vj > vi
        eq_before = (vj == vi) & (jg < ig)
        out_ref[0, 0, :] = jnp.sum((gt | eq_before).astype(jnp.int32), axis=1)

    rank = pl.pallas_call(
        body,
        grid=(B, T2 // bm),
        in_specs=[
            pl.BlockSpec((1, 1, bm), lambda b, i: (b, 0, i)),
            pl.BlockSpec((1, 1, T2), lambda b, i: (b, 0, 0)),
        ],
        out_specs=pl.BlockSpec((1, 1, bm), lambda b, i: (b, 0, i)),
        out_shape=jax.ShapeDtypeStruct((B, 1, T2), jnp.int32),
        compiler_params=pltpu.CompilerParams(
            dimension_semantics=("parallel", "arbitrary"),
        ),
    )(nm, nm)
    return rank


def _perm_scatter(rank_ref, rank_v, perm_v, b, T2, i16):
    pltpu.sync_copy(rank_ref.at[pl.ds(b * T2, T2)], rank_v)

    def pbody(i, carry):
        kv = plsc.load_gather(rank_v, [i16 + i * 16])
        plsc.store_scatter(perm_v, [kv], i16 + i * 16)
        return carry
    lax.fori_loop(0, T2 // 16, pbody, 0, unroll=8)


def _sc_slots(x_flat, rank_flat, nidx_flat, B, T2, C, r):
    mesh = plsc.VectorSubcoreMesh(core_axis_name="c", subcore_axis_name="s")

    @functools.partial(
        pl.kernel,
        out_type=[
            jax.ShapeDtypeStruct((B, _NW, 16, C), jnp.float32),
            jax.ShapeDtypeStruct((B, _NW, 16, C), jnp.float32),
            jax.ShapeDtypeStruct((B, _NW, 16), jnp.int32),
        ],
        mesh=mesh,
        compiler_params=pltpu.CompilerParams(needs_layout_passes=False),
        scratch_types=[
            pltpu.VMEM((T2,), jnp.int32),
            pltpu.VMEM((T2,), jnp.int32),
            pltpu.VMEM((T2,), jnp.int32),
            pltpu.VMEM((16,), jnp.int32),
            pltpu.VMEM((16,), jnp.int32),
            pltpu.VMEM((16, C), jnp.float32),
            pltpu.SemaphoreType.DMA,
        ],
    )
    def sc_kernel(x_ref, rank_ref, nidx_ref, srcg_ref, dstg_ref, dselp_ref,
                  rank_v, nidx_v, perm_v, idx16_v, dsel_v, rows16_v, sem):
        wid = lax.axis_index("s") * 2 + lax.axis_index("c")
        i16 = lax.broadcasted_iota(jnp.int32, (16,), 0)
        for b in range(B):
            base_x = b * 2 * T2
            pltpu.sync_copy(nidx_ref.at[pl.ds(b * T2, T2)], nidx_v)
            _perm_scatter(rank_ref, rank_v, perm_v, b, T2, i16)

            sbase = wid * _SPW
            permk = plsc.load_gather(perm_v, [i16 + sbase])
            dselv = plsc.load_gather(nidx_v, [permk])
            p408 = plsc.load_gather(perm_v, [jnp.full((16,), r - 1, jnp.int32)])
            d408 = plsc.load_gather(nidx_v, [p408])
            valid = (i16 < _SPW) & ((sbase + i16) < r)
            dselv = jnp.where(valid, dselv, d408)
            dsel_v[...] = dselv
            pltpu.sync_copy(dsel_v, dselp_ref.at[b, wid])
            idx16_v[...] = permk * 2 + base_x
            pltpu.async_copy(x_ref.at[idx16_v], rows16_v, sem).wait()
            pltpu.sync_copy(rows16_v, srcg_ref.at[b, wid])
            idx16_v[...] = dselv * 2 + (base_x + 1)
            pltpu.async_copy(x_ref.at[idx16_v], rows16_v, sem).wait()
            pltpu.sync_copy(rows16_v, dstg_ref.at[b, wid])

    return sc_kernel(x_flat, rank_flat, nidx_flat)


def _compact_merge(srcg, dstg, dselp, r):
    B = srcg.shape[0]
    S = _NW * 16
    C = srcg.shape[-1]
    src2 = srcg.reshape(B, S, C)
    dst2 = dstg.reshape(B, S, C)
    dsel2 = dselp.reshape(B, 1, S)

    def body(dsel_ref, src_ref, dst_ref, out_ref):
        dse = dsel_ref[0, 0, :]
        kp = lax.broadcasted_iota(jnp.int32, (S, S), 1)
        lane = kp & 15
        gslot = (kp >> 4) * _SPW + lane
        kvalid = (lane < _SPW) & (gslot < r)
        member = (dse[:, None] == dse[None, :]) & kvalid
        mf = member.astype(jnp.float32)
        upd = jnp.dot(mf, src_ref[0], preferred_element_type=jnp.float32,
                      precision=lax.Precision.HIGHEST)
        cnt = jnp.sum(mf, axis=1)
        out_ref[0] = (dst_ref[0] + upd) / (1.0 + cnt)[:, None]

    return pl.pallas_call(
        body,
        grid=(B,),
        in_specs=[
            pl.BlockSpec((1, 1, S), lambda b: (b, 0, 0)),
            pl.BlockSpec((1, S, C), lambda b: (b, 0, 0)),
            pl.BlockSpec((1, S, C), lambda b: (b, 0, 0)),
        ],
        out_specs=pl.BlockSpec((1, S, C), lambda b: (b, 0, 0)),
        out_shape=jax.ShapeDtypeStruct((B, S, C), jnp.float32),
    )(dsel2, src2, dst2)


def _sc_passthrough(out_ref_arg, x_flat, B, T2, C, tout, nunm):
    DPW = T2 // _NW
    mesh = plsc.VectorSubcoreMesh(core_axis_name="c", subcore_axis_name="s")

    @functools.partial(
        pl.kernel,
        out_type=[],
        mesh=mesh,
        compiler_params=pltpu.CompilerParams(needs_layout_passes=False),
        scratch_types=[
            pltpu.VMEM((128,), jnp.int32),
            pltpu.VMEM((128, C), jnp.float32),
            pltpu.SemaphoreType.DMA,
        ],
    )
    def sc_kernel(x_ref, out_ref, idx_v, rows_v, sem):
        wid = lax.axis_index("s") * 2 + lax.axis_index("c")
        i16 = lax.broadcasted_iota(jnp.int32, (16,), 0)
        for b in range(B):
            base_x = b * 2 * T2
            base_out = b * tout

            def dbody(i, carry):
                j = i * 16 + i16
                d = wid * DPW + j
                plsc.store_scatter(idx_v, [j], d * 2 + (base_x + 1))
                return carry
            lax.fori_loop(0, DPW // 16, dbody, 0, unroll=8)
            pltpu.async_copy(x_ref.at[idx_v], rows_v, sem).wait()
            def wbody(i, carry):
                j = i * 16 + i16
                plsc.store_scatter(idx_v, [j],
                                   base_out + nunm + wid * DPW + j)
                return carry
            lax.fori_loop(0, DPW // 16, wbody, 0, unroll=8)
            pltpu.async_copy(rows_v, out_ref.at[idx_v], sem).wait()

    sc_kernel(x_flat, out_ref_arg)


def _sc_finalize(out_ref_arg, x_flat, rank_flat, dsel512, dselp3, merged4,
                 B, T2, C, r, tout):
    nunm = T2 - r
    UPW = -(-nunm // _NW)
    mesh = plsc.VectorSubcoreMesh(core_axis_name="c", subcore_axis_name="s")

    @functools.partial(
        pl.kernel,
        out_type=[],
        mesh=mesh,
        compiler_params=pltpu.CompilerParams(needs_layout_passes=False),
        scratch_types=[
            pltpu.VMEM((T2,), jnp.int32),
            pltpu.VMEM((T2,), jnp.int32),
            pltpu.VMEM((512,), jnp.int32),
            pltpu.VMEM((128,), jnp.int32),
            pltpu.VMEM((128,), jnp.int32),
            pltpu.VMEM((128, C), jnp.float32),
            pltpu.SemaphoreType.DMA,
        ],
    )
    def sc_kernel(x_ref, rank_ref, dsel_ref, out_ref,
                  rank_v, perm_v, dsel_v, idx_v, widx_v,
                  rows_v, sem):
        wid = lax.axis_index("s") * 2 + lax.axis_index("c")
        i16 = lax.broadcasted_iota(jnp.int32, (16,), 0)
        for b in range(B):
            base_x = b * 2 * T2
            base_out = b * tout
            _perm_scatter(rank_ref, rank_v, perm_v, b, T2, i16)
            pltpu.sync_copy(dsel_ref.at[pl.ds(b * 512, 512)], dsel_v)

            s408 = (r - 1) // _SPW * 16 + (r - 1) % _SPW
            d408 = plsc.load_gather(dsel_v, [jnp.full((16,), s408, jnp.int32)])
            midx = base_out + nunm + d408

            def ubody(i, carry):
                j = i * 16 + i16
                uw = wid * UPW + j
                q = jnp.minimum(r + uw, T2 - 1)
                pv = plsc.load_gather(perm_v, [q])
                plsc.store_scatter(idx_v, [j], pv * 2 + base_x)
                ok = (j < UPW) & (uw < nunm)
                plsc.store_scatter(widx_v, [j],
                                   jnp.where(ok, base_out + uw, midx))
                return carry
            lax.fori_loop(0, 8, ubody, 0, unroll=8)
            pltpu.async_copy(x_ref.at[idx_v], rows_v, sem).wait()
            pltpu.async_copy(rows_v, out_ref.at[widx_v], sem).wait()


    sc_kernel(x_flat, rank_flat, dsel512, out_ref_arg)

    @functools.partial(
        pl.kernel,
        out_type=[],
        mesh=mesh,
        compiler_params=pltpu.CompilerParams(needs_layout_passes=False),
        scratch_types=[
            pltpu.VMEM((16,), jnp.int32),
            pltpu.VMEM((16,), jnp.int32),
            pltpu.VMEM((16, C), jnp.float32),
            pltpu.SemaphoreType.DMA,
        ],
    )
    def sc_scatter(dselp_ref, merged_ref, out_ref, dsel16_v, midx_v,
                   rows16_v, sem):
        wid = lax.axis_index("s") * 2 + lax.axis_index("c")
        for b in range(B):
            base_out = b * tout
            pltpu.sync_copy(dselp_ref.at[b, wid], dsel16_v)
            midx_v[...] = dsel16_v[...] + (base_out + nunm)
            pltpu.sync_copy(merged_ref.at[b, wid], rows16_v)
            pltpu.async_copy(rows16_v, out_ref.at[midx_v], sem).wait()

    sc_scatter(dselp3, merged4, out_ref_arg)


def kernel(x):
    B, T, C = x.shape
    T2 = T // 2
    r = math.floor(T - T * _R_RATIO)
    nunm = T2 - r
    tout = nunm + T2
    n = jnp.linalg.norm(x, axis=-1, keepdims=True)
    xn = x / jnp.clip(n, 1e-12)
    a_h, b_h = _deinterleave(xn, min(512, T2))

    nm, ni = _scores_maxarg(a_h, b_h, min(512, T2), min(1024, T2))
    rank = _ranks(nm, min(512, T2))

    x_flat = x.reshape(B * T, C)
    rank_flat = rank.reshape(B * T2)
    srcg, dstg, dselp = _sc_slots(x_flat, rank_flat, ni.reshape(B * T2),
                                  B, T2, C, r)
    merged = _compact_merge(srcg, dstg, dselp, r)

    out_ref = jax.new_ref(jnp.zeros((B * tout, C), jnp.float32))
    _sc_passthrough(out_ref, x_flat, B, T2, C, tout, nunm)
    _sc_finalize(out_ref, x_flat, rank_flat, dselp.reshape(B * 512), dselp,
                 merged.reshape(B, _NW, 16, C), B, T2, C, r, tout)
    return out_ref[...].reshape(B, tout, C)
